# baseline (speedup 1.0000x reference)
import jax
import jax.numpy as jnp
from jax import lax
from jax.experimental import pallas as pl
from jax.experimental.pallas import tpu as pltpu
from jax.experimental.pallas import tpu_sc as plsc

N = 20000
NPAD = 20480
BS = 80
NB = NPAD // BS
NG = NB // 16
BV = BS // 16
K = 300
KPAD = 320
THR = 0.7
NEG = -1.0e31
VTH = -1.0e30
EPS = 1e-9
L = 16
TILES = 16
CHUNK = NPAD // TILES
NBT = CHUNK // BS


def _iota():
    return lax.iota(jnp.int32, L)


def _nms_body(scores_hbm, ay1_hbm, ax1_hbm, ay2_hbm, ax2_hbm,
              dy_hbm, dx_hbm, dh_hbm, dw_hbm, hw_hbm, out_hbm,
              iay1, iax1, iay2, iax2, idy, idx_, idh, idw,
              y1l, x1l, y2l, x2l, scl, bml,
              spm_y1, spm_x1, spm_y2, spm_x2, spm_bm,
              scores_v, by1, bx1, by2, bx2, bm_v, bm2_v,
              acc_y1, acc_x1, acc_y2, acc_x2, acc_ar, outf, hw_v,
              sem_s, sem_1, sem_2, sem_3, sem_4, sem_5):
    c = lax.axis_index("c")
    s = lax.axis_index("s")
    iot = _iota()
    zf = jnp.zeros((L,), jnp.float32)
    negf = jnp.full((L,), NEG, jnp.float32)
    is0 = (c == 0) & (s == 0)

    @pl.when(is0)
    def _start_scores():
        pltpu.async_copy(scores_hbm, scores_v, sem_s)

    @pl.when(c == 0)
    def _decode():
        base = s * CHUNK
        sl_in = pl.ds(base, CHUNK)
        pltpu.sync_copy(ay1_hbm.at[sl_in], iay1)
        pltpu.sync_copy(ax1_hbm.at[sl_in], iax1)
        pltpu.sync_copy(ay2_hbm.at[sl_in], iay2)
        pltpu.sync_copy(ax2_hbm.at[sl_in], iax2)
        pltpu.sync_copy(dy_hbm.at[sl_in], idy)
        pltpu.sync_copy(dx_hbm.at[sl_in], idx_)
        pltpu.sync_copy(dh_hbm.at[sl_in], idh)
        pltpu.sync_copy(dw_hbm.at[sl_in], idw)
        pltpu.sync_copy(scores_hbm.at[sl_in], scl)
        pltpu.sync_copy(hw_hbm, hw_v)
        hh = hw_v[pl.ds(0, L)]
        ww = hw_v[pl.ds(L, L)]

        def dloop(j, _):
            sl = pl.ds(j * L, L)
            ay1 = iay1[sl]
            ax1 = iax1[sl]
            ay2 = iay2[sl]
            ax2 = iax2[sl]
            dy = idy[sl]
            dx = idx_[sl]
            dh = idh[sl]
            dw = idw[sl]
            ah = ay2 - ay1
            aw = ax2 - ax1
            acy = ay1 + 0.5 * ah
            acx = ax1 + 0.5 * aw
            pcy = dy * ah + acy
            pcx = dx * aw + acx
            ph = jnp.exp(dh) * ah
            pw = jnp.exp(dw) * aw
            z = jnp.float32(0.0)
            y1l[sl] = jnp.minimum(jnp.maximum(pcy - 0.5 * ph, z), hh)
            x1l[sl] = jnp.minimum(jnp.maximum(pcx - 0.5 * pw, z), ww)
            y2l[sl] = jnp.minimum(jnp.maximum(pcy + 0.5 * ph, z), hh)
            x2l[sl] = jnp.minimum(jnp.maximum(pcx + 0.5 * pw, z), ww)
            return 0

        lax.fori_loop(0, CHUNK // L, dloop, 0)

        def bloop(t, acc):
            def inner(j, mv):
                return jnp.maximum(mv, scl[pl.ds(t * BS + j * L, L)])

            mv = lax.fori_loop(0, BV, inner, negf)
            return jnp.where(iot == t, jnp.max(mv), acc)

        bml[pl.ds(0, L)] = lax.fori_loop(0, NBT, bloop, negf)

        dst = pl.ds(base, CHUNK)
        pltpu.sync_copy(y1l, spm_y1.at[dst])
        pltpu.sync_copy(x1l, spm_x1.at[dst])
        pltpu.sync_copy(y2l, spm_y2.at[dst])
        pltpu.sync_copy(x2l, spm_x2.at[dst])
        pltpu.sync_copy(bml, spm_bm.at[pl.ds(s * L, L)])

    plsc.subcore_barrier()

    @pl.when(is0)
    def _nms():
        cp1 = pltpu.async_copy(spm_y1, by1, sem_1)
        cp2 = pltpu.async_copy(spm_x1, bx1, sem_2)
        cp3 = pltpu.async_copy(spm_y2, by2, sem_3)
        cp4 = pltpu.async_copy(spm_x2, bx2, sem_4)
        cp5 = pltpu.async_copy(spm_bm, bm_v, sem_5)

        def zloop(k2, _):
            sl = pl.ds(k2 * L, L)
            acc_y1[sl] = zf
            acc_x1[sl] = zf
            acc_y2[sl] = zf
            acc_x2[sl] = zf
            acc_ar[sl] = zf
            return 0

        lax.fori_loop(0, KPAD // L, zloop, 0)

        def z2loop(k2, _):
            outf[pl.ds(k2 * L, L)] = zf
            return 0

        lax.fori_loop(0, (KPAD * 4) // L, z2loop, 0)

        cp5.wait()

        def gloop(g, acc):
            return jnp.where(iot == g, jnp.max(bm_v[pl.ds(g * L, L)]), acc)

        bm2_v[pl.ds(0, L)] = lax.fori_loop(0, NG, gloop, negf)

        cp1.wait()
        cp2.wait()
        cp3.wait()
        cp4.wait()
        pltpu.make_async_copy(scores_hbm, scores_v, sem_s).wait()

        BIGI = jnp.int32(2 ** 30)

        def cond(carry):
            cnt, alive = carry
            return (cnt < K) & (alive > 0)

        def body(carry):
            cnt, _ = carry

            g2 = bm2_v[pl.ds(0, L)]
            mv = jnp.full((L,), jnp.max(g2))
            gv = jnp.full((L,), jnp.min(jnp.where(g2 == mv, iot, BIGI)))
            alivev = mv > VTH
            chunk = plsc.load_gather(bm_v, [gv * L + iot])
            bv = gv * L + jnp.full((L,), jnp.min(jnp.where(chunk == mv, iot, BIGI)))

            bbase = bv * BS + iot
            rmv = negf
            smv = negf
            riv = jnp.zeros((L,), jnp.int32)
            for j in range(BV):
                v = plsc.load_gather(scores_v, [bbase + j * L])
                upd = v > rmv
                smv = jnp.maximum(smv, jnp.where(upd, rmv, v))
                riv = jnp.where(upd, j * L + iot, riv)
                rmv = jnp.where(upd, v, rmv)
            ibv = jnp.full((L,), jnp.min(jnp.where(rmv == mv, riv, BIGI)))
            gidxv = bv * BS + ibv
            rv = ibv - (ibv // L) * L

            plsc.store_scatter(scores_v, [gidxv], negf, mask=iot == 0)
            nbmv = jnp.full((L,), jnp.max(jnp.where(iot == rv, smv, rmv)))
            nchunk = jnp.where(iot == (bv - gv * L), nbmv, chunk)
            plsc.store_scatter(bm_v, [bv], nbmv, mask=iot == 0)
            plsc.store_scatter(bm2_v, [gv], jnp.full((L,), jnp.max(nchunk)),
                               mask=iot == 0)

            cy1 = plsc.load_gather(by1, [gidxv])
            cx1 = plsc.load_gather(bx1, [gidxv])
            cy2 = plsc.load_gather(by2, [gidxv])
            cx2 = plsc.load_gather(bx2, [gidxv])
            car = jnp.maximum(cy2 - cy1, 0.0) * jnp.maximum(cx2 - cx1, 0.0)

            def one(ofs, mx):
                sl = pl.ds(ofs, L)
                iy1 = jnp.maximum(acc_y1[sl], cy1)
                ix1 = jnp.maximum(acc_x1[sl], cx1)
                iy2 = jnp.minimum(acc_y2[sl], cy2)
                ix2 = jnp.minimum(acc_x2[sl], cx2)
                inter = jnp.maximum(iy2 - iy1, 0.0) * jnp.maximum(ix2 - ix1, 0.0)
                return jnp.maximum(mx, inter / (acc_ar[sl] + car - inter + EPS))

            def chk(k4, accs):
                a0, a1, a2, a3 = accs
                base = k4 * (4 * L)
                return (one(base, a0), one(base + L, a1),
                        one(base + 2 * L, a2), one(base + 3 * L, a3))

            n4 = (cnt + (4 * L - 1)) // (4 * L)
            a0, a1, a2, a3 = lax.fori_loop(0, n4, chk, (zf, zf, zf, zf))
            mxv = jnp.maximum(jnp.maximum(a0, a1), jnp.maximum(a2, a3))
            acceptv = (jnp.full((L,), jnp.max(mxv)) <= THR) & alivev

            am = acceptv & (iot == 0)
            cv = jnp.full((L,), cnt, jnp.int32)
            plsc.store_scatter(acc_y1, [cv], cy1, mask=am)
            plsc.store_scatter(acc_x1, [cv], cx1, mask=am)
            plsc.store_scatter(acc_y2, [cv], cy2, mask=am)
            plsc.store_scatter(acc_x2, [cv], cx2, mask=am)
            plsc.store_scatter(acc_ar, [cv], car, mask=am)
            row = jnp.where(iot == 0, cy1,
                            jnp.where(iot == 1, cx1,
                                      jnp.where(iot == 2, cy2, cx2)))
            plsc.store_scatter(outf, [4 * cv + iot], row, mask=acceptv & (iot < 4))

            flags = jnp.where(acceptv, 1, 0) + jnp.where(alivev, 2, 0)
            fl = jnp.max(flags)
            return (cnt + (fl & 1), fl // 2)

        lax.while_loop(cond, body, (jnp.int32(0), jnp.int32(1)))
        pltpu.sync_copy(outf.at[pl.ds(0, 4 * K)], out_hbm)


_mesh = plsc.VectorSubcoreMesh(core_axis_name="c", subcore_axis_name="s")

_nms_call = pl.kernel(
    _nms_body,
    out_type=jax.ShapeDtypeStruct((4 * K,), jnp.float32),
    mesh=_mesh,
    compiler_params=pltpu.CompilerParams(needs_layout_passes=False),
    scratch_types=(
        [pltpu.VMEM((CHUNK,), jnp.float32)] * 8 +
        [pltpu.VMEM((CHUNK,), jnp.float32)] * 4 +
        [pltpu.VMEM((CHUNK,), jnp.float32)] +
        [pltpu.VMEM((L,), jnp.float32)] +
        [pltpu.VMEM_SHARED((NPAD,), jnp.float32)] * 4 +
        [pltpu.VMEM_SHARED((NB,), jnp.float32)] +
        [pltpu.VMEM((NPAD,), jnp.float32)] * 5 +
        [pltpu.VMEM((NB,), jnp.float32)] +
        [pltpu.VMEM((L,), jnp.float32)] +
        [pltpu.VMEM((KPAD,), jnp.float32)] * 5 +
        [pltpu.VMEM((KPAD * 4,), jnp.float32)] +
        [pltpu.VMEM((2 * L,), jnp.float32)] +
        [pltpu.SemaphoreType.DMA] * 6
    ),
)


@jax.jit
def kernel(rpn_scores, rpn_coordinates, anchors, image_shape):
    img_h = image_shape[0].astype(jnp.float32)
    img_w = image_shape[1].astype(jnp.float32)
    hw = jnp.concatenate([jnp.full((L,), img_h), jnp.full((L,), img_w)])
    padf = jnp.zeros((NPAD - N,), jnp.float32)
    scores_p = jnp.concatenate(
        [rpn_scores.astype(jnp.float32), jnp.full((NPAD - N,), NEG, jnp.float32)])
    a = anchors.astype(jnp.float32)
    d = rpn_coordinates.astype(jnp.float32)
    cols = [jnp.concatenate([a[:, i], padf]) for i in range(4)] + \
           [jnp.concatenate([d[:, i], padf]) for i in range(4)]
    out = _nms_call(scores_p, *cols, hw)
    return lax.stop_gradient(out.reshape(K, 4))


# speedup vs baseline: 1.1599x; 1.1599x over previous
import jax
import jax.numpy as jnp
from jax import lax
from jax.experimental import pallas as pl
from jax.experimental.pallas import tpu as pltpu
from jax.experimental.pallas import tpu_sc as plsc

N = 20000
NPAD = 20480
BS = 80
NB = NPAD // BS
NG = NB // 16
BV = BS // 16
K = 300
KPAD = 320
THR = 0.7
NEG = -1.0e31
VTH = -1.0e30
EPS = 1e-9
L = 16
TILES = 16
CHUNK = NPAD // TILES
NBT = CHUNK // BS


def _iota():
    return lax.iota(jnp.int32, L)


def _nms_body(scores_hbm, ay1_hbm, ax1_hbm, ay2_hbm, ax2_hbm,
              dy_hbm, dx_hbm, dh_hbm, dw_hbm, hw_hbm, out_hbm,
              iay1, iax1, iay2, iax2, idy, idx_, idh, idw,
              y1l, x1l, y2l, x2l, scl, bml,
              spm_y1, spm_x1, spm_y2, spm_x2, spm_bm,
              scores_v, by1, bx1, by2, bx2, bm_v, bm2_v,
              acc_y1, acc_x1, acc_y2, acc_x2, acc_ar, outf, hw_v,
              sem_s, sem_1, sem_2, sem_3, sem_4, sem_5):
    c = lax.axis_index("c")
    s = lax.axis_index("s")
    iot = _iota()
    zf = jnp.zeros((L,), jnp.float32)
    negf = jnp.full((L,), NEG, jnp.float32)
    is0 = (c == 0) & (s == 0)

    @pl.when(is0)
    def _start_scores():
        pltpu.async_copy(scores_hbm, scores_v, sem_s)

    @pl.when(c == 0)
    def _decode():
        base = s * CHUNK
        sl_in = pl.ds(base, CHUNK)
        pltpu.sync_copy(ay1_hbm.at[sl_in], iay1)
        pltpu.sync_copy(ax1_hbm.at[sl_in], iax1)
        pltpu.sync_copy(ay2_hbm.at[sl_in], iay2)
        pltpu.sync_copy(ax2_hbm.at[sl_in], iax2)
        pltpu.sync_copy(dy_hbm.at[sl_in], idy)
        pltpu.sync_copy(dx_hbm.at[sl_in], idx_)
        pltpu.sync_copy(dh_hbm.at[sl_in], idh)
        pltpu.sync_copy(dw_hbm.at[sl_in], idw)
        pltpu.sync_copy(scores_hbm.at[sl_in], scl)
        pltpu.sync_copy(hw_hbm, hw_v)
        hh = hw_v[pl.ds(0, L)]
        ww = hw_v[pl.ds(L, L)]

        def dloop(j, _):
            sl = pl.ds(j * L, L)
            ay1 = iay1[sl]
            ax1 = iax1[sl]
            ay2 = iay2[sl]
            ax2 = iax2[sl]
            dy = idy[sl]
            dx = idx_[sl]
            dh = idh[sl]
            dw = idw[sl]
            ah = ay2 - ay1
            aw = ax2 - ax1
            acy = ay1 + 0.5 * ah
            acx = ax1 + 0.5 * aw
            pcy = dy * ah + acy
            pcx = dx * aw + acx
            ph = jnp.exp(dh) * ah
            pw = jnp.exp(dw) * aw
            z = jnp.float32(0.0)
            y1l[sl] = jnp.minimum(jnp.maximum(pcy - 0.5 * ph, z), hh)
            x1l[sl] = jnp.minimum(jnp.maximum(pcx - 0.5 * pw, z), ww)
            y2l[sl] = jnp.minimum(jnp.maximum(pcy + 0.5 * ph, z), hh)
            x2l[sl] = jnp.minimum(jnp.maximum(pcx + 0.5 * pw, z), ww)
            return 0

        lax.fori_loop(0, CHUNK // L, dloop, 0)

        def bloop(t, acc):
            def inner(j, mv):
                return jnp.maximum(mv, scl[pl.ds(t * BS + j * L, L)])

            mv = lax.fori_loop(0, BV, inner, negf)
            return jnp.where(iot == t, jnp.max(mv), acc)

        bml[pl.ds(0, L)] = lax.fori_loop(0, NBT, bloop, negf)

        dst = pl.ds(base, CHUNK)
        pltpu.sync_copy(y1l, spm_y1.at[dst])
        pltpu.sync_copy(x1l, spm_x1.at[dst])
        pltpu.sync_copy(y2l, spm_y2.at[dst])
        pltpu.sync_copy(x2l, spm_x2.at[dst])
        pltpu.sync_copy(bml, spm_bm.at[pl.ds(s * L, L)])

    plsc.subcore_barrier()

    @pl.when(is0)
    def _nms():
        cp1 = pltpu.async_copy(spm_y1, by1, sem_1)
        cp2 = pltpu.async_copy(spm_x1, bx1, sem_2)
        cp3 = pltpu.async_copy(spm_y2, by2, sem_3)
        cp4 = pltpu.async_copy(spm_x2, bx2, sem_4)
        cp5 = pltpu.async_copy(spm_bm, bm_v, sem_5)

        def zloop(k2, _):
            sl = pl.ds(k2 * L, L)
            acc_y1[sl] = zf
            acc_x1[sl] = zf
            acc_y2[sl] = zf
            acc_x2[sl] = zf
            acc_ar[sl] = zf
            return 0

        lax.fori_loop(0, KPAD // L, zloop, 0)

        def z2loop(k2, _):
            outf[pl.ds(k2 * L, L)] = zf
            return 0

        lax.fori_loop(0, (KPAD * 4) // L, z2loop, 0)

        cp5.wait()

        def gloop(g, acc):
            return jnp.where(iot == g, jnp.max(bm_v[pl.ds(g * L, L)]), acc)

        bm2_v[pl.ds(0, L)] = lax.fori_loop(0, NG, gloop, negf)

        cp1.wait()
        cp2.wait()
        cp3.wait()
        cp4.wait()
        pltpu.make_async_copy(scores_hbm, scores_v, sem_s).wait()

        BIGI = jnp.int32(2 ** 30)

        def cond(carry):
            cnt, alive = carry
            return (cnt < K) & (alive > 0)

        iotf = iot.astype(jnp.float32)
        stripf = iotf * BV
        BIGF = jnp.float32(1.0e9)

        def body(carry):
            cnt, _ = carry

            g2 = bm2_v[pl.ds(0, L)]
            mv = jnp.full((L,), jnp.max(g2))
            gv = plsc.all_reduce_ffs(g2 == mv)
            alivev = mv > VTH
            chunk = plsc.load_gather(bm_v, [gv * L + iot])
            lane1 = plsc.all_reduce_ffs(chunk == mv)
            bv = gv * L + lane1

            bbase = bv * BS + iot * BV
            rmv = negf
            smv = negf
            rivf = zf
            for j in range(BV):
                v = plsc.load_gather(scores_v, [bbase + j])
                upd = v > rmv
                smv = jnp.maximum(smv, jnp.where(upd, rmv, v))
                rivf = jnp.where(upd, stripf + j, rivf)
                rmv = jnp.where(upd, v, rmv)
            mask2 = rmv == mv
            ibv = jnp.full((L,), jnp.min(jnp.where(mask2, rivf, BIGF))).astype(jnp.int32)
            rvv = plsc.all_reduce_ffs(mask2)
            gidxv = bv * BS + ibv

            plsc.store_scatter(scores_v, [gidxv], negf, mask=iot == 0)
            nbmv = jnp.full((L,), jnp.max(jnp.where(iot == rvv, smv, rmv)))
            nchunk = jnp.where(iot == lane1, nbmv, chunk)
            plsc.store_scatter(bm_v, [bv], nbmv, mask=iot == 0)
            plsc.store_scatter(bm2_v, [gv], jnp.full((L,), jnp.max(nchunk)),
                               mask=iot == 0)

            cy1 = plsc.load_gather(by1, [gidxv])
            cx1 = plsc.load_gather(bx1, [gidxv])
            cy2 = plsc.load_gather(by2, [gidxv])
            cx2 = plsc.load_gather(bx2, [gidxv])
            car = jnp.maximum(cy2 - cy1, 0.0) * jnp.maximum(cx2 - cx1, 0.0)

            def one(ofs, mx):
                sl = pl.ds(ofs, L)
                iy1 = jnp.maximum(acc_y1[sl], cy1)
                ix1 = jnp.maximum(acc_x1[sl], cx1)
                iy2 = jnp.minimum(acc_y2[sl], cy2)
                ix2 = jnp.minimum(acc_x2[sl], cx2)
                inter = jnp.maximum(iy2 - iy1, 0.0) * jnp.maximum(ix2 - ix1, 0.0)
                return jnp.maximum(mx, inter / (acc_ar[sl] + car - inter + EPS))

            def chk(k4, accs):
                a0, a1, a2, a3 = accs
                base = k4 * (4 * L)
                return (one(base, a0), one(base + L, a1),
                        one(base + 2 * L, a2), one(base + 3 * L, a3))

            n4 = (cnt + (4 * L - 1)) // (4 * L)
            a0, a1, a2, a3 = lax.fori_loop(0, n4, chk, (zf, zf, zf, zf))
            mxv = jnp.maximum(jnp.maximum(a0, a1), jnp.maximum(a2, a3))
            acceptv = (jnp.full((L,), jnp.max(mxv)) <= THR) & alivev

            am = acceptv & (iot == 0)
            cv = jnp.full((L,), cnt, jnp.int32)
            plsc.store_scatter(acc_y1, [cv], cy1, mask=am)
            plsc.store_scatter(acc_x1, [cv], cx1, mask=am)
            plsc.store_scatter(acc_y2, [cv], cy2, mask=am)
            plsc.store_scatter(acc_x2, [cv], cx2, mask=am)
            plsc.store_scatter(acc_ar, [cv], car, mask=am)
            row = jnp.where(iot == 0, cy1,
                            jnp.where(iot == 1, cx1,
                                      jnp.where(iot == 2, cy2, cx2)))
            plsc.store_scatter(outf, [4 * cv + iot], row, mask=acceptv & (iot < 4))

            flags = jnp.where(acceptv, 1.0, 0.0) + jnp.where(alivev, 2.0, 0.0)
            fl = jnp.max(flags).astype(jnp.int32)
            return (cnt + (fl & 1), fl // 2)

        lax.while_loop(cond, body, (jnp.int32(0), jnp.int32(1)))
        pltpu.sync_copy(outf.at[pl.ds(0, 4 * K)], out_hbm)


_mesh = plsc.VectorSubcoreMesh(core_axis_name="c", subcore_axis_name="s")

_nms_call = pl.kernel(
    _nms_body,
    out_type=jax.ShapeDtypeStruct((4 * K,), jnp.float32),
    mesh=_mesh,
    compiler_params=pltpu.CompilerParams(needs_layout_passes=False),
    scratch_types=(
        [pltpu.VMEM((CHUNK,), jnp.float32)] * 8 +
        [pltpu.VMEM((CHUNK,), jnp.float32)] * 4 +
        [pltpu.VMEM((CHUNK,), jnp.float32)] +
        [pltpu.VMEM((L,), jnp.float32)] +
        [pltpu.VMEM_SHARED((NPAD,), jnp.float32)] * 4 +
        [pltpu.VMEM_SHARED((NB,), jnp.float32)] +
        [pltpu.VMEM((NPAD,), jnp.float32)] * 5 +
        [pltpu.VMEM((NB,), jnp.float32)] +
        [pltpu.VMEM((L,), jnp.float32)] +
        [pltpu.VMEM((KPAD,), jnp.float32)] * 5 +
        [pltpu.VMEM((KPAD * 4,), jnp.float32)] +
        [pltpu.VMEM((2 * L,), jnp.float32)] +
        [pltpu.SemaphoreType.DMA] * 6
    ),
)


@jax.jit
def kernel(rpn_scores, rpn_coordinates, anchors, image_shape):
    img_h = image_shape[0].astype(jnp.float32)
    img_w = image_shape[1].astype(jnp.float32)
    hw = jnp.concatenate([jnp.full((L,), img_h), jnp.full((L,), img_w)])
    padf = jnp.zeros((NPAD - N,), jnp.float32)
    scores_p = jnp.concatenate(
        [rpn_scores.astype(jnp.float32), jnp.full((NPAD - N,), NEG, jnp.float32)])
    a = anchors.astype(jnp.float32)
    d = rpn_coordinates.astype(jnp.float32)
    cols = [jnp.concatenate([a[:, i], padf]) for i in range(4)] + \
           [jnp.concatenate([d[:, i], padf]) for i in range(4)]
    out = _nms_call(scores_p, *cols, hw)
    return lax.stop_gradient(out.reshape(K, 4))
